# Initial kernel scaffold; baseline (speedup 1.0000x reference)
import jax
import jax.numpy as jnp
from jax.experimental import pallas as pl


def kernel(embedding_table, position_embeds, input_ids):
    raise NotImplementedError("write your pallas kernel here")


# speedup vs baseline: 3.4585x; 3.4585x over previous
import functools

import jax
import jax.numpy as jnp
from jax import lax
from jax.experimental import pallas as pl
from jax.experimental.pallas import tpu as pltpu
from jax.experimental.pallas import tpu_sc as plsc

VOCAB = 100000
EMBED = 64
MAX_POS = 200
BATCH = 4096
SEQ = 200

NC = 2
NS = 16
NW = NC * NS
LANES = 16

ROWS = BATCH * SEQ
R_PER_W = ROWS // NW
CHUNK = 400
NCHUNK = R_PER_W // CHUNK
GSUB = 4
MSUB = CHUNK // GSUB
REPS = CHUNK // MAX_POS


def _emb_body(table_hbm, pos_hbm, ids_hbm, out_hbm, pos_v, idx_v, rows_v, sem):
    cid = lax.axis_index("c")
    sid = lax.axis_index("s")
    wid = sid * NC + cid

    pltpu.sync_copy(pos_hbm, pos_v)

    def chunk_body(c, carry):
        row0 = (wid * NCHUNK + c) * CHUNK
        pltpu.sync_copy(ids_hbm.at[wid * NCHUNK + c], idx_v)
        cps = [
            pltpu.async_copy(
                table_hbm.at[idx_v.at[g]],
                rows_v.at[pl.ds(g * MSUB, MSUB)],
                sem,
            )
            for g in range(GSUB)
        ]
        for cp in cps:
            cp.wait()

        def add_body(p, acc):
            for j in range(EMBED // LANES):
                pv = pos_v[p, pl.ds(j * LANES, LANES)]
                for rep in range(REPS):
                    r = rep * MAX_POS + p
                    rows_v[r, pl.ds(j * LANES, LANES)] = (
                        rows_v[r, pl.ds(j * LANES, LANES)] + pv
                    )
            return acc

        lax.fori_loop(0, MAX_POS, add_body, 0)
        pltpu.sync_copy(rows_v, out_hbm.at[pl.ds(row0, CHUNK)])
        return carry

    lax.fori_loop(0, NCHUNK, chunk_body, 0)


@jax.jit
def _emb(table, pos2d, ids3d):
    mesh = plsc.VectorSubcoreMesh(core_axis_name="c", subcore_axis_name="s")
    return pl.kernel(
        _emb_body,
        out_type=jax.ShapeDtypeStruct((ROWS, EMBED), jnp.float32),
        mesh=mesh,
        scratch_types=[
            pltpu.VMEM((MAX_POS, EMBED), jnp.float32),
            pltpu.VMEM((GSUB, MSUB), jnp.int32),
            pltpu.VMEM((CHUNK, EMBED), jnp.float32),
            pltpu.SemaphoreType.DMA,
        ],
        compiler_params=pltpu.CompilerParams(use_tc_tiling_on_sc=False),
    )(table, pos2d, ids3d)


def kernel(embedding_table, position_embeds, input_ids):
    ids3d = input_ids.astype(jnp.int32).reshape(NW * NCHUNK, GSUB, MSUB)
    pos2d = position_embeds.reshape(MAX_POS, EMBED)
    out = _emb(embedding_table, pos2d, ids3d)
    return out.reshape(BATCH, SEQ, EMBED)
